# baseline (speedup 1.0000x reference)
import functools

import jax
import jax.numpy as jnp
from jax import lax
from jax.experimental import pallas as pl
from jax.experimental.pallas import tpu as pltpu
from jax.experimental.pallas import tpu_sc as plsc

BATCH = 16384
DIM = 64

_info = plsc.get_sparse_core_info()
_NC = _info.num_cores
_NS = _info.num_subcores
_NW = _NC * _NS
_BPW = BATCH // _NW
_G = 16


def _build():
    mesh = plsc.VectorSubcoreMesh(core_axis_name="c", subcore_axis_name="s")

    @functools.partial(
        pl.kernel,
        mesh=mesh,
        out_type=(
            jax.ShapeDtypeStruct((BATCH, DIM), jnp.float32),
            jax.ShapeDtypeStruct((BATCH, DIM), jnp.float32),
        ),
        scratch_types=[
            pltpu.VMEM((_BPW,), jnp.int32),
            pltpu.VMEM((_BPW,), jnp.int32),
            pltpu.SemaphoreType.DMA,
        ],
    )
    def k(adduct_hbm, instr_hbm, atab_hbm, itab_hbm, outa_hbm, outb_hbm,
          aidx_v, iidx_v, sem):
        wid = lax.axis_index("s") * _NC + lax.axis_index("c")
        base = wid * _BPW
        pltpu.sync_copy(adduct_hbm.at[pl.ds(base, _BPW)], aidx_v)
        pltpu.sync_copy(instr_hbm.at[pl.ds(base, _BPW)], iidx_v)

        def grp(g, _):
            av = aidx_v[pl.ds(g * _G, _G)]
            iv = iidx_v[pl.ds(g * _G, _G)]
            copies = []
            for i in range(_G):
                r = base + g * _G + i
                copies.append(pltpu.async_copy(
                    atab_hbm.at[pl.ds(av[i], 1), :],
                    outa_hbm.at[pl.ds(r, 1), :], sem))
                copies.append(pltpu.async_copy(
                    itab_hbm.at[pl.ds(iv[i], 1), :],
                    outb_hbm.at[pl.ds(r, 1), :], sem))
            for cp in copies:
                cp.wait()
            return ()

        lax.fori_loop(0, _BPW // _G, grp, ())

    return k


_sc_kernel = _build()


def kernel(adduct, instrument_type, adduct_table, instrument_type_table):
    outa, outb = _sc_kernel(adduct, instrument_type,
                            adduct_table, instrument_type_table)
    return jnp.concatenate([outa, outb], axis=1)


# speedup vs baseline: 4.6352x; 4.6352x over previous
import functools

import jax
import jax.numpy as jnp
from jax import lax
from jax.experimental import pallas as pl
from jax.experimental.pallas import tpu as pltpu
from jax.experimental.pallas import tpu_sc as plsc

BATCH = 16384
DIM = 64
ODIM = 2 * DIM

_info = plsc.get_sparse_core_info()
_NC = _info.num_cores
_NS = _info.num_subcores
_NW = _NC * _NS
_BPW = BATCH // _NW
_CH = 128
_NCHUNK = _BPW // _CH
_CPP = 2
_PR = _CPP * _CH
_NPASS = _NCHUNK // _CPP


def _pad_right_block(in_ref, out_ref):
    out_ref[...] = jnp.concatenate(
        [in_ref[...], jnp.zeros_like(in_ref)], axis=1)


def _pad_left_block(in_ref, out_ref):
    out_ref[...] = jnp.concatenate(
        [jnp.zeros_like(in_ref), in_ref[...]], axis=1)


def _tc_pad(table, left):
    rows = table.shape[0]
    blk = 2048
    grid = (rows + blk - 1) // blk
    return pl.pallas_call(
        _pad_left_block if left else _pad_right_block,
        grid=(grid,),
        in_specs=[pl.BlockSpec((blk, DIM), lambda i: (i, 0))],
        out_specs=pl.BlockSpec((blk, ODIM), lambda i: (i, 0)),
        out_shape=jax.ShapeDtypeStruct((rows, ODIM), jnp.float32),
    )(table)


def _build():
    mesh = plsc.VectorSubcoreMesh(core_axis_name="c", subcore_axis_name="s")

    @functools.partial(
        pl.kernel,
        mesh=mesh,
        out_type=jax.ShapeDtypeStruct((BATCH, ODIM), jnp.float32),
        scratch_types=[
            pltpu.VMEM((_NCHUNK, _CH), jnp.int32),
            pltpu.VMEM((_NCHUNK, _CH), jnp.int32),
            pltpu.VMEM((_PR, ODIM), jnp.float32),
            pltpu.VMEM((_PR, ODIM), jnp.float32),
            pltpu.SemaphoreType.DMA,
        ],
    )
    def k(adduct_hbm, instr_hbm, apad_hbm, ipad_hbm, out_hbm,
          aidx_v, iidx_v, a_v, b_v, sem):
        wid = lax.axis_index("s") * _NC + lax.axis_index("c")
        base = wid * _BPW
        row0 = wid * _NCHUNK
        pltpu.sync_copy(adduct_hbm.at[pl.ds(row0, _NCHUNK), :], aidx_v)
        pltpu.sync_copy(instr_hbm.at[pl.ds(row0, _NCHUNK), :], iidx_v)
        for p in range(_NPASS):
            copies = []
            for j in range(_CPP):
                c = p * _CPP + j
                copies.append(pltpu.async_copy(
                    apad_hbm.at[aidx_v.at[c]],
                    a_v.at[pl.ds(j * _CH, _CH)], sem))
                copies.append(pltpu.async_copy(
                    ipad_hbm.at[iidx_v.at[c]],
                    b_v.at[pl.ds(j * _CH, _CH)], sem))
            for cp in copies:
                cp.wait()

            def addrow(r, _):
                for k16 in range(ODIM // 16):
                    sl = pl.ds(k16 * 16, 16)
                    a_v[r, sl] = a_v[r, sl] + b_v[r, sl]
                return ()

            lax.fori_loop(0, _PR, addrow, ())
            pltpu.sync_copy(a_v, out_hbm.at[pl.ds(base + p * _PR, _PR), :])

    return k


_sc_kernel = _build()


def kernel(adduct, instrument_type, adduct_table, instrument_type_table):
    apad = _tc_pad(adduct_table, left=False)
    ipad = _tc_pad(instrument_type_table, left=True)
    adduct2 = adduct.reshape(_NW * _NCHUNK, _CH)
    instr2 = instrument_type.reshape(_NW * _NCHUNK, _CH)
    return _sc_kernel(adduct2, instr2, apad, ipad)


# speedup vs baseline: 5.6025x; 1.2087x over previous
import functools

import jax
import jax.numpy as jnp
from jax import lax
from jax.experimental import pallas as pl
from jax.experimental.pallas import tpu as pltpu
from jax.experimental.pallas import tpu_sc as plsc

BATCH = 16384
DIM = 64
ODIM = 2 * DIM

_info = plsc.get_sparse_core_info()
_NC = _info.num_cores
_NS = _info.num_subcores
_NW = _NC * _NS
_BPW = BATCH // _NW
_CH = 128
_NCHUNK = _BPW // _CH
_CPP = 2
_PR = _CPP * _CH
_NPASS = _NCHUNK // _CPP


def _pad_right_block(in_ref, out_ref):
    out_ref[:, :DIM] = in_ref[...]


def _pad_left_block(in_ref, out_ref):
    out_ref[:, DIM:] = in_ref[...]


def _tc_pad(table, left):
    rows = table.shape[0]
    blk = min(8192, rows + 7 - (rows + 7) % 8)
    grid = (rows + blk - 1) // blk
    return pl.pallas_call(
        _pad_left_block if left else _pad_right_block,
        grid=(grid,),
        in_specs=[pl.BlockSpec((blk, DIM), lambda i: (i, 0))],
        out_specs=pl.BlockSpec((blk, ODIM), lambda i: (i, 0)),
        out_shape=jax.ShapeDtypeStruct((rows, ODIM), jnp.float32),
    )(table)


def _build():
    mesh = plsc.VectorSubcoreMesh(core_axis_name="c", subcore_axis_name="s")

    @functools.partial(
        pl.kernel,
        mesh=mesh,
        out_type=jax.ShapeDtypeStruct((BATCH, ODIM), jnp.float32),
        scratch_types=[
            pltpu.VMEM((_NCHUNK, _CH), jnp.int32),
            pltpu.VMEM((_NCHUNK, _CH), jnp.int32),
            pltpu.VMEM((_PR, ODIM), jnp.float32),
            pltpu.VMEM((_PR, ODIM), jnp.float32),
            pltpu.SemaphoreType.DMA,
        ],
    )
    def k(adduct_hbm, instr_hbm, apad_hbm, ipad_hbm, out_hbm,
          aidx_v, iidx_v, a_v, b_v, sem):
        wid = lax.axis_index("s") * _NC + lax.axis_index("c")
        base = wid * _BPW
        row0 = wid * _NCHUNK
        pltpu.sync_copy(adduct_hbm.at[pl.ds(row0, _NCHUNK), :], aidx_v)
        pltpu.sync_copy(instr_hbm.at[pl.ds(row0, _NCHUNK), :], iidx_v)
        for p in range(_NPASS):
            copies = []
            for j in range(_CPP):
                c = p * _CPP + j
                copies.append(pltpu.async_copy(
                    apad_hbm.at[aidx_v.at[c]],
                    a_v.at[pl.ds(j * _CH, _CH)], sem))
                copies.append(pltpu.async_copy(
                    ipad_hbm.at[iidx_v.at[c]],
                    b_v.at[pl.ds(j * _CH, _CH)], sem))
            for cp in copies:
                cp.wait()

            def mergerow(r, _):
                for k16 in range(DIM // 16):
                    sl = pl.ds(DIM + k16 * 16, 16)
                    a_v[r, sl] = b_v[r, sl]
                return ()

            lax.fori_loop(0, _PR, mergerow, ())
            pltpu.sync_copy(a_v, out_hbm.at[pl.ds(base + p * _PR, _PR), :])

    return k


_sc_kernel = _build()


def kernel(adduct, instrument_type, adduct_table, instrument_type_table):
    apad = _tc_pad(adduct_table, left=False)
    ipad = _tc_pad(instrument_type_table, left=True)
    adduct2 = adduct.reshape(_NW * _NCHUNK, _CH)
    instr2 = instrument_type.reshape(_NW * _NCHUNK, _CH)
    return _sc_kernel(adduct2, instr2, apad, ipad)
